# baseline (speedup 1.0000x reference)
import functools

import jax
import jax.numpy as jnp
from jax import lax
from jax.experimental import pallas as pl
from jax.experimental import layout as jex_layout
from jax.experimental.pallas import tpu as pltpu
from jax.experimental.pallas import tpu_sc as plsc

B0, B1 = 16384, 50
NUM_TOKENS = B0 * B1
EMBED = 32

NC = 2
NS = 16
NW = NC * NS
B_PER_W = NUM_TOKENS // NW
CHUNK = 1024
N_CHUNKS = B_PER_W // CHUNK


def _make_emb_kernel():
    mesh = plsc.VectorSubcoreMesh(core_axis_name="c", subcore_axis_name="s")

    @functools.partial(
        pl.kernel,
        mesh=mesh,
        out_type=jax.ShapeDtypeStruct((NUM_TOKENS, EMBED), jnp.float32),
        scratch_types=[
            pltpu.VMEM((CHUNK,), jnp.int32),
            pltpu.VMEM((CHUNK, EMBED), jnp.float32),
            pltpu.SemaphoreType.DMA,
        ],
        compiler_params=pltpu.CompilerParams(use_tc_tiling_on_sc=False),
    )
    def emb_kernel(idx_hbm, table_hbm, out_hbm, idx_v, rows_v, sem):
        wid = lax.axis_index("s") * NC + lax.axis_index("c")
        base = wid * B_PER_W

        def body(i, _):
            off = base + i * CHUNK
            pltpu.sync_copy(idx_hbm.at[pl.ds(off, CHUNK)], idx_v)
            pltpu.async_copy(table_hbm.at[idx_v], rows_v, sem).wait()
            pltpu.sync_copy(rows_v, out_hbm.at[pl.ds(off, CHUNK)])
            return 0

        lax.fori_loop(0, N_CHUNKS, body, 0)

    return emb_kernel


_emb = _make_emb_kernel()

def _impl(token, table):
    flat = token.reshape(-1)
    out = _emb(flat, table)
    return out.reshape(B0, B1, EMBED)


@functools.cache
def _jitted_for(dev):
    fmt = jex_layout.Format(
        jex_layout.Layout(major_to_minor=(2, 1, 0), tiling=((8,),)),
        jax.sharding.SingleDeviceSharding(dev),
    )
    return jax.jit(_impl, out_shardings=fmt)


def kernel(token, table):
    return _jitted_for(jax.devices()[0])(token, table)


# speedup vs baseline: 1.6292x; 1.6292x over previous
import functools

import jax
import jax.numpy as jnp
from jax import lax
from jax.experimental import pallas as pl
from jax.experimental.pallas import tpu as pltpu
from jax.experimental.pallas import tpu_sc as plsc

B0, B1 = 16384, 50
NUM_TOKENS = B0 * B1
EMBED = 32

NC = 2
NS = 16
NW = NC * NS
ROWS_PER_W = B0 // NW
RCHUNK = 32
N_CHUNKS = ROWS_PER_W // RCHUNK
CHUNK = RCHUNK * B1


def _make_emb_kernel():
    mesh = plsc.VectorSubcoreMesh(core_axis_name="c", subcore_axis_name="s")

    @functools.partial(
        pl.kernel,
        mesh=mesh,
        out_type=jax.ShapeDtypeStruct((B0, B1, EMBED), jnp.float32),
        scratch_types=[
            pltpu.VMEM((CHUNK,), jnp.int32),
            pltpu.VMEM((CHUNK, EMBED), jnp.float32),
            pltpu.SemaphoreType.DMA,
            pltpu.SemaphoreType.DMA,
        ],
        compiler_params=pltpu.CompilerParams(use_tc_tiling_on_sc=False),
    )
    def emb_kernel(idx_hbm, table_hbm, out_hbm, idx_v, rows_v, gsem, osem):
        wid = lax.axis_index("s") * NC + lax.axis_index("c")
        row_base = wid * ROWS_PER_W

        def body(i, _):
            row_off = row_base + i * RCHUNK
            off = row_off * B1
            pltpu.sync_copy(idx_hbm.at[pl.ds(off, CHUNK)], idx_v)
            pltpu.async_copy(table_hbm.at[idx_v], rows_v, gsem).wait()
            handles = [
                pltpu.async_copy(
                    rows_v.at[pl.ds(j * B1, B1)], out_hbm.at[row_off + j], osem
                )
                for j in range(RCHUNK)
            ]
            for h in handles:
                h.wait()
            return 0

        lax.fori_loop(0, N_CHUNKS, body, 0)

    return emb_kernel


_emb = _make_emb_kernel()


@jax.jit
def kernel(token, table):
    flat = token.reshape(-1)
    return _emb(flat, table)
